# Initial kernel scaffold; baseline (speedup 1.0000x reference)
import jax
import jax.numpy as jnp
from jax.experimental import pallas as pl


def kernel(loc_data, priors, targets):
    raise NotImplementedError("write your pallas kernel here")


# speedup vs baseline: 97.1027x; 97.1027x over previous
import functools

import jax
import jax.numpy as jnp
import numpy as np
from jax import lax
from jax.experimental import pallas as pl
from jax.experimental.pallas import tpu as pltpu
from jax.experimental.pallas import tpu_sc as plsc

_B, _K, _G = 32, 8192, 16
_L = 16
_CHUNKS = _K // _L
_CLIP = 256.0
_MAXN = 2.0 * _CLIP
_EPS = float(np.finfo(np.float32).eps)

_mesh = plsc.VectorSubcoreMesh(
    core_axis_name="c", subcore_axis_name="s", num_cores=2, num_subcores=16
)


_SCRATCH = [
    pltpu.VMEM((2 * _K,), jnp.float32),
    pltpu.VMEM((_K,), jnp.float32),
    pltpu.VMEM((3 * _G,), jnp.float32),
    pltpu.VMEM((2 * _L,), jnp.float32),
    pltpu.VMEM((2 * _L,), jnp.float32),
    pltpu.VMEM((2 * _L,), jnp.float32),
]


def _tile_body(loc_hbm, pri_hbm, tgt_hbm, out_hbm,
               loc_v, pri_v, tgt_v, s_v, e_v, ob_v):
    cid = lax.axis_index("c")
    sid = lax.axis_index("s")
    wid = sid * 2 + cid

    pltpu.sync_copy(loc_hbm.at[wid], loc_v)
    pltpu.sync_copy(pri_hbm, pri_v)
    pltpu.sync_copy(tgt_hbm.at[wid], tgt_v)

    iota = lax.iota(jnp.int32, _L)

    starts = plsc.load_gather(tgt_v, [3 * iota])
    ends = plsc.load_gather(tgt_v, [3 * iota + 1])
    s_v[pl.ds(_L, _L)] = starts * _CLIP
    e_v[pl.ds(_L, _L)] = ends * _CLIP

    sg = []
    eg = []
    for g in range(_G):
        gidx = jnp.full((_L,), _L + g, jnp.int32)
        sg.append(plsc.load_gather(s_v, [gidx]))
        eg.append(plsc.load_gather(e_v, [gidx]))

    def chunk(i, carry):
        sacc, cacc = carry
        c256 = pri_v[pl.ds(i * _L, _L)] * _CLIP
        best_a = jnp.full((_L,), _MAXN, jnp.float32)
        best_l = jnp.zeros((_L,), jnp.float32)
        best_r = jnp.zeros((_L,), jnp.float32)
        for g in range(_G):
            lt = c256 - sg[g]
            rt = eg[g] - c256
            a = lt + rt
            a = jnp.where(jnp.minimum(lt, rt) < 0.0, _MAXN, a)
            upd = a < best_a
            best_a = jnp.where(upd, a, best_a)
            best_l = jnp.where(upd, lt, best_l)
            best_r = jnp.where(upd, rt, best_r)
        kidx = 2 * (i * _L + iota)
        p_l = jnp.maximum(plsc.load_gather(loc_v, [kidx]), 0.0)
        p_r = jnp.maximum(plsc.load_gather(loc_v, [kidx + 1]), 0.0)
        inter = jnp.minimum(p_l, best_l) + jnp.minimum(p_r, best_r)
        union = (best_l + best_r) + (p_l + p_r) - inter
        iou = inter / jnp.maximum(union, _EPS)
        pos = best_a < _MAXN
        sacc = sacc + jnp.where(pos, 1.0 - iou, 0.0)
        cacc = cacc + jnp.where(pos, 1.0, 0.0)
        return sacc, cacc

    z = jnp.zeros((_L,), jnp.float32)
    sacc, cacc = lax.fori_loop(0, _CHUNKS, chunk, (z, z))
    ob_v[pl.ds(0, _L)] = sacc
    ob_v[pl.ds(_L, _L)] = cacc
    pltpu.sync_copy(ob_v, out_hbm.at[wid])


_loc_loss_partials = pl.kernel(
    _tile_body,
    out_type=jax.ShapeDtypeStruct((_B, 2 * _L), jnp.float32),
    mesh=_mesh,
    scratch_types=_SCRATCH,
    compiler_params=pltpu.CompilerParams(needs_layout_passes=False),
)


def kernel(loc_data, priors, targets):
    parts = _loc_loss_partials(loc_data.reshape(_B, 2 * _K), priors.reshape(_K),
                               targets.reshape(_B, 3 * _G))
    masked_sum = jnp.sum(parts[:, :_L])
    num_pos = jnp.sum(parts[:, _L:])
    p = jnp.maximum(num_pos, 1.0)
    return jnp.where(num_pos > 0, masked_sum / p, 0.0) / p


# speedup vs baseline: 155.7209x; 1.6037x over previous
import functools

import jax
import jax.numpy as jnp
import numpy as np
from jax import lax
from jax.experimental import pallas as pl
from jax.experimental.pallas import tpu as pltpu
from jax.experimental.pallas import tpu_sc as plsc

_B, _K, _G = 32, 8192, 16
_L = 16
_CHUNKS = _K // _L
_CLIP = 256.0
_MAXN = 2.0 * _CLIP
_EPS = float(np.finfo(np.float32).eps)

_mesh = plsc.VectorSubcoreMesh(
    core_axis_name="c", subcore_axis_name="s", num_cores=2, num_subcores=16
)


_SCRATCH = [
    pltpu.VMEM((_K // 128, 2, 128), jnp.float32),
    pltpu.VMEM((_K,), jnp.float32),
    pltpu.VMEM((3 * _G,), jnp.float32),
    pltpu.VMEM((2 * _L,), jnp.float32),
    pltpu.VMEM((2 * _L,), jnp.float32),
    pltpu.VMEM((2 * _L,), jnp.float32),
]


def _tile_body(loc_hbm, pri_hbm, tgt_hbm, out_hbm,
               loc_v, pri_v, tgt_v, s_v, e_v, ob_v):
    cid = lax.axis_index("c")
    sid = lax.axis_index("s")
    wid = sid * 2 + cid

    pltpu.sync_copy(loc_hbm.at[wid], loc_v)
    pltpu.sync_copy(pri_hbm, pri_v)
    pltpu.sync_copy(tgt_hbm.at[wid], tgt_v)

    iota = lax.iota(jnp.int32, _L)

    starts = plsc.load_gather(tgt_v, [3 * iota])
    ends = plsc.load_gather(tgt_v, [3 * iota + 1])
    s_v[pl.ds(_L, _L)] = starts * _CLIP
    e_v[pl.ds(_L, _L)] = ends * _CLIP

    sg = []
    eg = []
    for g in range(_G):
        gidx = jnp.full((_L,), _L + g, jnp.int32)
        sg.append(plsc.load_gather(s_v, [gidx]))
        eg.append(plsc.load_gather(e_v, [gidx]))

    def chunk(i, carry):
        sacc, cacc = carry
        c256 = pri_v[pl.ds(i * _L, _L)] * _CLIP
        best_a = jnp.full((_L,), _MAXN, jnp.float32)
        best_l = jnp.zeros((_L,), jnp.float32)
        best_r = jnp.zeros((_L,), jnp.float32)
        for g in range(_G):
            lt = c256 - sg[g]
            rt = eg[g] - c256
            a = lt + rt
            a = jnp.where(jnp.minimum(lt, rt) < 0.0, _MAXN, a)
            upd = a < best_a
            best_a = jnp.where(upd, a, best_a)
            best_l = jnp.where(upd, lt, best_l)
            best_r = jnp.where(upd, rt, best_r)
        kt = lax.shift_right_logical(i, 3)
        j = (i & 7) * _L
        p_l = jnp.maximum(loc_v[kt, 0, pl.ds(j, _L)], 0.0)
        p_r = jnp.maximum(loc_v[kt, 1, pl.ds(j, _L)], 0.0)
        inter = jnp.minimum(p_l, best_l) + jnp.minimum(p_r, best_r)
        union = (best_l + best_r) + (p_l + p_r) - inter
        iou = inter / jnp.maximum(union, _EPS)
        pos = best_a < _MAXN
        sacc = sacc + jnp.where(pos, 1.0 - iou, 0.0)
        cacc = cacc + jnp.where(pos, 1.0, 0.0)
        return sacc, cacc

    z = jnp.zeros((_L,), jnp.float32)
    sacc, cacc = lax.fori_loop(0, _CHUNKS, chunk, (z, z))
    ob_v[pl.ds(0, _L)] = sacc
    ob_v[pl.ds(_L, _L)] = cacc
    pltpu.sync_copy(ob_v, out_hbm.at[wid])


_loc_loss_partials = pl.kernel(
    _tile_body,
    out_type=jax.ShapeDtypeStruct((_B, 2 * _L), jnp.float32),
    mesh=_mesh,
    scratch_types=_SCRATCH,
    compiler_params=pltpu.CompilerParams(needs_layout_passes=False),
)


def kernel(loc_data, priors, targets):
    loc_blk = loc_data.reshape(_B, _K // 128, 128, 2).transpose(0, 1, 3, 2)
    parts = _loc_loss_partials(loc_blk, priors.reshape(_K),
                               targets.reshape(_B, 3 * _G))
    masked_sum = jnp.sum(parts[:, :_L])
    num_pos = jnp.sum(parts[:, _L:])
    p = jnp.maximum(num_pos, 1.0)
    return jnp.where(num_pos > 0, masked_sum / p, 0.0) / p


# speedup vs baseline: 168.9268x; 1.0848x over previous
import functools

import jax
import jax.numpy as jnp
import numpy as np
from jax import lax
from jax.experimental import pallas as pl
from jax.experimental.pallas import tpu as pltpu
from jax.experimental.pallas import tpu_sc as plsc

_B, _K, _G = 32, 8192, 16
_L = 16
_CHUNKS = _K // _L
_CLIP = 256.0
_MAXN = 2.0 * _CLIP
_EPS = float(np.finfo(np.float32).eps)

_mesh = plsc.VectorSubcoreMesh(
    core_axis_name="c", subcore_axis_name="s", num_cores=2, num_subcores=16
)


_SCRATCH = [
    pltpu.VMEM((_K // 128, 2, 128), jnp.float32),
    pltpu.VMEM((_K,), jnp.float32),
    pltpu.VMEM((3 * _G,), jnp.float32),
    pltpu.VMEM((2 * _L,), jnp.float32),
    pltpu.VMEM((2 * _L,), jnp.float32),
    pltpu.VMEM((2 * _L,), jnp.float32),
    pltpu.VMEM((_K,), jnp.float32),
    pltpu.VMEM((_K,), jnp.float32),
    pltpu.VMEM((_K,), jnp.float32),
]


def _tile_body(loc_hbm, pri_hbm, tgt_hbm, out_hbm,
               loc_v, pri_v, tgt_v, s_v, e_v, ob_v, ba_v, bl_v, br_v):
    cid = lax.axis_index("c")
    sid = lax.axis_index("s")
    wid = sid * 2 + cid

    pltpu.sync_copy(loc_hbm.at[wid], loc_v)
    pltpu.sync_copy(pri_hbm, pri_v)
    pltpu.sync_copy(tgt_hbm.at[wid], tgt_v)

    iota = lax.iota(jnp.int32, _L)

    starts = plsc.load_gather(tgt_v, [3 * iota])
    ends = plsc.load_gather(tgt_v, [3 * iota + 1])
    s_v[pl.ds(_L, _L)] = starts * _CLIP
    e_v[pl.ds(_L, _L)] = ends * _CLIP

    sg = []
    eg = []
    for g in range(_G):
        gidx = jnp.full((_L,), _L + g, jnp.int32)
        sg.append(plsc.load_gather(s_v, [gidx]))
        eg.append(plsc.load_gather(e_v, [gidx]))

    def scan_g(c256, g_lo, g_hi, best):
        best_a, best_l, best_r = best
        for g in range(g_lo, g_hi):
            lt = c256 - sg[g]
            rt = eg[g] - c256
            a = lt + rt
            a = jnp.where(jnp.minimum(lt, rt) < 0.0, _MAXN, a)
            upd = a < best_a
            best_a = jnp.where(upd, a, best_a)
            best_l = jnp.where(upd, lt, best_l)
            best_r = jnp.where(upd, rt, best_r)
        return best_a, best_l, best_r

    def chunk_a(i, carry):
        c256 = pri_v[pl.ds(i * _L, _L)] * _CLIP
        best_a = jnp.full((_L,), _MAXN, jnp.float32)
        z16 = jnp.zeros((_L,), jnp.float32)
        best_a, best_l, best_r = scan_g(c256, 0, _G // 2, (best_a, z16, z16))
        ba_v[pl.ds(i * _L, _L)] = best_a
        bl_v[pl.ds(i * _L, _L)] = best_l
        br_v[pl.ds(i * _L, _L)] = best_r
        return carry

    lax.fori_loop(0, _CHUNKS, chunk_a, 0)

    def chunk_b(i, carry):
        sacc, cacc = carry
        c256 = pri_v[pl.ds(i * _L, _L)] * _CLIP
        best = (ba_v[pl.ds(i * _L, _L)],
                bl_v[pl.ds(i * _L, _L)],
                br_v[pl.ds(i * _L, _L)])
        best_a, best_l, best_r = scan_g(c256, _G // 2, _G, best)
        kt = lax.shift_right_logical(i, 3)
        j = (i & 7) * _L
        p_l = jnp.maximum(loc_v[kt, 0, pl.ds(j, _L)], 0.0)
        p_r = jnp.maximum(loc_v[kt, 1, pl.ds(j, _L)], 0.0)
        inter = jnp.minimum(p_l, best_l) + jnp.minimum(p_r, best_r)
        union = (best_l + best_r) + (p_l + p_r) - inter
        iou = inter / jnp.maximum(union, _EPS)
        pos = best_a < _MAXN
        sacc = sacc + jnp.where(pos, 1.0 - iou, 0.0)
        cacc = cacc + jnp.where(pos, 1.0, 0.0)
        return sacc, cacc

    z = jnp.zeros((_L,), jnp.float32)
    sacc, cacc = lax.fori_loop(0, _CHUNKS, chunk_b, (z, z))
    ob_v[pl.ds(0, _L)] = sacc
    ob_v[pl.ds(_L, _L)] = cacc
    pltpu.sync_copy(ob_v, out_hbm.at[wid])


_loc_loss_partials = pl.kernel(
    _tile_body,
    out_type=jax.ShapeDtypeStruct((_B, 2 * _L), jnp.float32),
    mesh=_mesh,
    scratch_types=_SCRATCH,
    compiler_params=pltpu.CompilerParams(needs_layout_passes=False),
)


def kernel(loc_data, priors, targets):
    loc_blk = loc_data.reshape(_B, _K // 128, 128, 2).transpose(0, 1, 3, 2)
    parts = _loc_loss_partials(loc_blk, priors.reshape(_K),
                               targets.reshape(_B, 3 * _G))
    masked_sum = jnp.sum(parts[:, :_L])
    num_pos = jnp.sum(parts[:, _L:])
    p = jnp.maximum(num_pos, 1.0)
    return jnp.where(num_pos > 0, masked_sum / p, 0.0) / p


# speedup vs baseline: 213.8185x; 1.2657x over previous
import jax
import jax.numpy as jnp
import numpy as np
from jax import lax
from jax.experimental import pallas as pl
from jax.experimental.pallas import tpu as pltpu
from jax.experimental.pallas import tpu_sc as plsc

_B, _K, _G = 32, 8192, 16
_L = 16
_CHUNKS = _K // _L
_CLIP = 256.0
_MAXN = 2.0 * _CLIP
_EPS = float(np.finfo(np.float32).eps)
_BIG = np.uint32(0x7FFFFFFF)

_mesh = plsc.VectorSubcoreMesh(
    core_axis_name="c", subcore_axis_name="s", num_cores=2, num_subcores=16
)


_SCRATCH = [
    pltpu.VMEM((_K // 128, 2, 128), jnp.float32),
    pltpu.VMEM((_K,), jnp.float32),
    pltpu.VMEM((3 * _G,), jnp.float32),
    pltpu.VMEM((2 * _L,), jnp.float32),
    pltpu.VMEM((2 * _L,), jnp.float32),
    pltpu.VMEM((2 * _L,), jnp.int32),
    pltpu.VMEM((2 * _L,), jnp.float32),
    pltpu.VMEM((_K,), jnp.int32),
    pltpu.SemaphoreType.DMA,
]


def _tile_body(loc_hbm, pri_hbm, tgt_hbm, out_hbm,
               loc_v, pri_v, tgt_v, s_v, e_v, k_v, ob_v, bk_v, sem):
    cid = lax.axis_index("c")
    sid = lax.axis_index("s")
    wid = sid * 2 + cid

    loc_cp = pltpu.async_copy(loc_hbm.at[wid], loc_v, sem)
    pltpu.sync_copy(tgt_hbm.at[wid], tgt_v)
    pltpu.sync_copy(pri_hbm, pri_v)

    iota = lax.iota(jnp.int32, _L)

    starts = plsc.load_gather(tgt_v, [3 * iota])
    ends = plsc.load_gather(tgt_v, [3 * iota + 1])
    s256 = starts * _CLIP
    e256 = ends * _CLIP
    area_bits = plsc.bitcast(e256 - s256, jnp.int32)
    keys = (area_bits & np.int32(~0xF)) | iota
    s_v[pl.ds(_L, _L)] = s256
    e_v[pl.ds(_L, _L)] = e256
    k_v[pl.ds(_L, _L)] = keys

    sg = []
    eg = []
    kg = []
    for g in range(_G):
        gidx = jnp.full((_L,), _L + g, jnp.int32)
        sg.append(plsc.load_gather(s_v, [gidx]))
        eg.append(plsc.load_gather(e_v, [gidx]))
        kg.append(plsc.bitcast(plsc.load_gather(k_v, [gidx]), jnp.uint32))

    def scan_g(c256, g_lo, g_hi, best):
        for g in range(g_lo, g_hi):
            hit = (c256 >= sg[g]) & (c256 <= eg[g])
            best = jnp.minimum(best, jnp.where(hit, kg[g], _BIG))
        return best

    big = jnp.full((_L,), _BIG, jnp.uint32)

    def chunk_a(i, carry):
        c256 = pri_v[pl.ds(i * _L, _L)] * _CLIP
        best = scan_g(c256, 0, _G // 2, big)
        bk_v[pl.ds(i * _L, _L)] = plsc.bitcast(best, jnp.int32)
        return carry

    lax.fori_loop(0, _CHUNKS, chunk_a, 0)
    loc_cp.wait()

    def chunk_b(i, carry):
        sacc, cacc = carry
        c256 = pri_v[pl.ds(i * _L, _L)] * _CLIP
        best = scan_g(c256, _G // 2, _G, big)
        prev = plsc.bitcast(bk_v[pl.ds(i * _L, _L)], jnp.uint32)
        best = jnp.minimum(best, prev)
        gsel = (plsc.bitcast(best, jnp.int32) & 0xF) + _L
        t_l = plsc.load_gather(s_v, [gsel])
        t_r = plsc.load_gather(e_v, [gsel])
        t_l = c256 - t_l
        t_r = t_r - c256
        kt = lax.shift_right_logical(i, 3)
        j = (i & 7) * _L
        p_l = jnp.maximum(loc_v[kt, 0, pl.ds(j, _L)], 0.0)
        p_r = jnp.maximum(loc_v[kt, 1, pl.ds(j, _L)], 0.0)
        inter = jnp.minimum(p_l, t_l) + jnp.minimum(p_r, t_r)
        union = (t_l + t_r) + (p_l + p_r) - inter
        iou = inter / jnp.maximum(union, _EPS)
        pos = best < _BIG
        sacc = sacc + jnp.where(pos, 1.0 - iou, 0.0)
        cacc = cacc + jnp.where(pos, 1.0, 0.0)
        return sacc, cacc

    z = jnp.zeros((_L,), jnp.float32)
    sacc, cacc = lax.fori_loop(0, _CHUNKS, chunk_b, (z, z))
    ob_v[pl.ds(0, _L)] = sacc
    ob_v[pl.ds(_L, _L)] = cacc
    pltpu.sync_copy(ob_v, out_hbm.at[wid])


_loc_loss_partials = pl.kernel(
    _tile_body,
    out_type=jax.ShapeDtypeStruct((_B, 2 * _L), jnp.float32),
    mesh=_mesh,
    scratch_types=_SCRATCH,
    compiler_params=pltpu.CompilerParams(needs_layout_passes=False),
)


def kernel(loc_data, priors, targets):
    loc_blk = loc_data.reshape(_B, _K // 128, 128, 2).transpose(0, 1, 3, 2)
    parts = _loc_loss_partials(loc_blk, priors.reshape(_K),
                               targets.reshape(_B, 3 * _G))
    masked_sum = jnp.sum(parts[:, :_L])
    num_pos = jnp.sum(parts[:, _L:])
    p = jnp.maximum(num_pos, 1.0)
    return jnp.where(num_pos > 0, masked_sum / p, 0.0) / p


# speedup vs baseline: 214.1639x; 1.0016x over previous
import jax
import jax.numpy as jnp
import numpy as np
from jax import lax
from jax.experimental import pallas as pl
from jax.experimental.pallas import tpu as pltpu
from jax.experimental.pallas import tpu_sc as plsc

_B, _K, _G = 32, 8192, 16
_L = 16
_CHUNKS = _K // _L
_CLIP = 256.0
_MAXN = 2.0 * _CLIP
_EPS = float(np.finfo(np.float32).eps)
_BIG = np.uint32(0x7FFFFFFF)

_mesh = plsc.VectorSubcoreMesh(
    core_axis_name="c", subcore_axis_name="s", num_cores=2, num_subcores=16
)


_SCRATCH = [
    pltpu.VMEM((_K // 128, 2, 128), jnp.float32),
    pltpu.VMEM((_K,), jnp.float32),
    pltpu.VMEM((3 * _G,), jnp.float32),
    pltpu.VMEM((2 * _L,), jnp.float32),
    pltpu.VMEM((2 * _L,), jnp.float32),
    pltpu.VMEM((2 * _L,), jnp.int32),
    pltpu.VMEM((2 * _L,), jnp.float32),
    pltpu.VMEM((_K,), jnp.int32),
    pltpu.SemaphoreType.DMA,
]


def _tile_body(loc_hbm, pri_hbm, tgt_hbm, out_hbm,
               loc_v, pri_v, tgt_v, s_v, e_v, k_v, ob_v, bk_v, sem):
    cid = lax.axis_index("c")
    sid = lax.axis_index("s")
    wid = sid * 2 + cid

    loc_cp = pltpu.async_copy(loc_hbm.at[wid], loc_v, sem)
    pltpu.sync_copy(tgt_hbm.at[wid], tgt_v)
    pltpu.sync_copy(pri_hbm, pri_v)

    iota = lax.iota(jnp.int32, _L)

    starts = plsc.load_gather(tgt_v, [3 * iota])
    ends = plsc.load_gather(tgt_v, [3 * iota + 1])
    s256 = starts * _CLIP
    e256 = ends * _CLIP
    area_bits = plsc.bitcast(e256 - s256, jnp.int32)
    keys = (area_bits & np.int32(~0xF)) | iota
    s_v[pl.ds(_L, _L)] = s256
    e_v[pl.ds(_L, _L)] = e256
    k_v[pl.ds(_L, _L)] = keys

    sg = []
    eg = []
    kg = []
    for g in range(_G):
        gidx = jnp.full((_L,), _L + g, jnp.int32)
        sg.append(plsc.load_gather(s_v, [gidx]))
        eg.append(plsc.load_gather(e_v, [gidx]))
        kg.append(plsc.bitcast(plsc.load_gather(k_v, [gidx]), jnp.uint32))

    def scan_g(c256, g_lo, g_hi, best):
        for g in range(g_lo, g_hi):
            hit = (c256 >= sg[g]) & (c256 <= eg[g])
            best = jnp.minimum(best, jnp.where(hit, kg[g], _BIG))
        return best

    big = jnp.full((_L,), _BIG, jnp.uint32)

    def chunk_a(i, carry):
        c256 = pri_v[pl.ds(i * _L, _L)] * _CLIP
        best = scan_g(c256, 0, _G // 2, big)
        bk_v[pl.ds(i * _L, _L)] = plsc.bitcast(best, jnp.int32)
        return carry

    lax.fori_loop(0, _CHUNKS, chunk_a, 0)
    loc_cp.wait()

    def chunk_b(i, carry):
        sacc, cacc = carry
        c256 = pri_v[pl.ds(i * _L, _L)] * _CLIP
        best = scan_g(c256, _G // 2, _G, big)
        prev = plsc.bitcast(bk_v[pl.ds(i * _L, _L)], jnp.uint32)
        best = jnp.minimum(best, prev)
        gsel = (plsc.bitcast(best, jnp.int32) & 0xF) + _L
        t_l = plsc.load_gather(s_v, [gsel])
        t_r = plsc.load_gather(e_v, [gsel])
        t_l = c256 - t_l
        t_r = t_r - c256
        kt = lax.shift_right_logical(i, 3)
        j = (i & 7) * _L
        p_l = jnp.maximum(loc_v[kt, 0, pl.ds(j, _L)], 0.0)
        p_r = jnp.maximum(loc_v[kt, 1, pl.ds(j, _L)], 0.0)
        inter = jnp.minimum(p_l, t_l) + jnp.minimum(p_r, t_r)
        union = (t_l + t_r) + (p_l + p_r) - inter
        iou = inter / jnp.maximum(union, _EPS)
        pos = best < _BIG
        sacc = sacc + jnp.where(pos, 1.0 - iou, 0.0)
        cacc = cacc + jnp.where(pos, 1.0, 0.0)
        return sacc, cacc

    z = jnp.zeros((_L,), jnp.float32)
    sacc, cacc = lax.fori_loop(0, _CHUNKS, chunk_b, (z, z))
    ob_v[pl.ds(0, _L)] = sacc
    ob_v[pl.ds(_L, _L)] = cacc
    pltpu.sync_copy(ob_v, out_hbm.at[wid])


_loc_loss_partials = pl.kernel(
    _tile_body,
    out_type=jax.ShapeDtypeStruct((_B, 2 * _L), jnp.float32),
    mesh=_mesh,
    scratch_types=_SCRATCH,
    compiler_params=pltpu.CompilerParams(needs_layout_passes=False, skip_device_barrier=True),
)


def kernel(loc_data, priors, targets):
    loc_blk = loc_data.reshape(_B, _K // 128, 128, 2).transpose(0, 1, 3, 2)
    parts = _loc_loss_partials(loc_blk, priors.reshape(_K),
                               targets.reshape(_B, 3 * _G))
    masked_sum = jnp.sum(parts[:, :_L])
    num_pos = jnp.sum(parts[:, _L:])
    p = jnp.maximum(num_pos, 1.0)
    return jnp.where(num_pos > 0, masked_sum / p, 0.0) / p


# speedup vs baseline: 214.6343x; 1.0022x over previous
import jax
import jax.numpy as jnp
import numpy as np
from jax import lax
from jax.experimental import pallas as pl
from jax.experimental.pallas import tpu as pltpu
from jax.experimental.pallas import tpu_sc as plsc

_B, _K, _G = 32, 8192, 16
_L = 16
_CHUNKS = _K // _L
_CLIP = 256.0
_EPS = float(np.finfo(np.float32).eps)
_BIG = np.uint32(0x7FFFFFFF)

_mesh = plsc.VectorSubcoreMesh(
    core_axis_name="c", subcore_axis_name="s", num_cores=2, num_subcores=16
)


_SCRATCH = [
    pltpu.VMEM((_K // 128, 2, 128), jnp.float32),
    pltpu.VMEM((_K,), jnp.float32),
    pltpu.VMEM((3 * _G,), jnp.float32),
    pltpu.VMEM((2 * _L,), jnp.float32),
    pltpu.VMEM((2 * _L,), jnp.float32),
    pltpu.VMEM((2 * _L,), jnp.int32),
    pltpu.VMEM((2 * _L,), jnp.float32),
    pltpu.VMEM((_K,), jnp.int32),
    pltpu.SemaphoreType.DMA,
]


def _tile_body(loc_hbm, pri_hbm, tgt_hbm, out_hbm,
               loc_v, pri_v, tgt_v, s_v, e_v, k_v, ob_v, bk_v, sem):
    cid = lax.axis_index("c")
    sid = lax.axis_index("s")
    wid = sid * 2 + cid

    loc_cp = pltpu.async_copy(loc_hbm.at[wid], loc_v, sem)
    pltpu.sync_copy(tgt_hbm.at[wid], tgt_v)
    pltpu.sync_copy(pri_hbm, pri_v)

    iota = lax.iota(jnp.int32, _L)

    starts = plsc.load_gather(tgt_v, [3 * iota])
    ends = plsc.load_gather(tgt_v, [3 * iota + 1])
    s256 = starts * _CLIP
    e256 = ends * _CLIP
    area_bits = plsc.bitcast(e256 - s256, jnp.int32)
    keys = (area_bits & np.int32(~0xF)) | iota
    s_v[pl.ds(_L, _L)] = s256
    e_v[pl.ds(_L, _L)] = e256
    k_v[pl.ds(_L, _L)] = keys

    sg = []
    eg = []
    kg = []
    for g in range(_G):
        gidx = jnp.full((_L,), _L + g, jnp.int32)
        sg.append(plsc.load_gather(s_v, [gidx]))
        eg.append(plsc.load_gather(e_v, [gidx]))
        kg.append(plsc.bitcast(plsc.load_gather(k_v, [gidx]), jnp.uint32))

    def scan_g(c256, g_lo, g_hi, best):
        for g in range(g_lo, g_hi):
            hit = (c256 >= sg[g]) & (c256 <= eg[g])
            best = jnp.minimum(best, jnp.where(hit, kg[g], _BIG))
        return best

    big = jnp.full((_L,), _BIG, jnp.uint32)

    def chunk_a(i, carry):
        c256 = pri_v[pl.ds(i * _L, _L)] * _CLIP
        best = scan_g(c256, 0, _G // 2, big)
        bk_v[pl.ds(i * _L, _L)] = plsc.bitcast(best, jnp.int32)
        return carry

    lax.fori_loop(0, _CHUNKS, chunk_a, 0)
    loc_cp.wait()

    def chunk_b(i, carry):
        sacc, cacc = carry
        c256 = pri_v[pl.ds(i * _L, _L)] * _CLIP
        best = scan_g(c256, _G // 2, _G, big)
        prev = plsc.bitcast(bk_v[pl.ds(i * _L, _L)], jnp.uint32)
        best = jnp.minimum(best, prev)
        gsel = (plsc.bitcast(best, jnp.int32) & 0xF) + _L
        t_l = plsc.load_gather(s_v, [gsel])
        t_r = plsc.load_gather(e_v, [gsel])
        t_l = c256 - t_l
        t_r = t_r - c256
        kt = lax.shift_right_logical(i, 3)
        j = (i & 7) * _L
        p_l = jnp.maximum(loc_v[kt, 0, pl.ds(j, _L)], 0.0)
        p_r = jnp.maximum(loc_v[kt, 1, pl.ds(j, _L)], 0.0)
        inter = jnp.minimum(p_l, t_l) + jnp.minimum(p_r, t_r)
        union = (t_l + t_r) + (p_l + p_r) - inter
        iou = inter / jnp.maximum(union, _EPS)
        pos = best < _BIG
        sacc = sacc + jnp.where(pos, 1.0 - iou, 0.0)
        cacc = cacc + jnp.where(pos, 1.0, 0.0)
        return sacc, cacc

    z = jnp.zeros((_L,), jnp.float32)
    sacc, cacc = lax.fori_loop(0, _CHUNKS, chunk_b, (z, z))
    ob_v[pl.ds(0, _L)] = sacc
    ob_v[pl.ds(_L, _L)] = cacc
    pltpu.sync_copy(ob_v, out_hbm.at[wid])


_loc_loss_partials = pl.kernel(
    _tile_body,
    out_type=jax.ShapeDtypeStruct((_B, 2 * _L), jnp.float32),
    mesh=_mesh,
    scratch_types=_SCRATCH,
    compiler_params=pltpu.CompilerParams(needs_layout_passes=False),
)


def kernel(loc_data, priors, targets):
    loc_blk = loc_data.reshape(_B, _K // 128, 128, 2).transpose(0, 1, 3, 2)
    parts = _loc_loss_partials(loc_blk, priors.reshape(_K),
                               targets.reshape(_B, 3 * _G))
    masked_sum = jnp.sum(parts[:, :_L])
    num_pos = jnp.sum(parts[:, _L:])
    p = jnp.maximum(num_pos, 1.0)
    return jnp.where(num_pos > 0, masked_sum / p, 0.0) / p
